# baseline (speedup 1.0000x reference)
import functools

import jax
import jax.numpy as jnp
from jax import lax
from jax.experimental import pallas as pl
from jax.experimental.pallas import tpu as pltpu
from jax.experimental.pallas import tpu_sc as plsc

B = 4096
S = 200
S1 = S + 1
D = 64
COND_DIM = 128
VOCAB = 1000000

NC = 2
NS = 16
NW = NC * NS
BPW = B // NW

C0 = 128
C1 = S - C0


def _dense_body(ce_ref, wc_ref, bc_ref, t_ref, wt_ref, bt_ref,
                cond_ref, temb_ref):
    cond_ref[...] = (
        jnp.dot(ce_ref[...], wc_ref[...], preferred_element_type=jnp.float32)
        + bc_ref[...]
    )
    temb_ref[...] = t_ref[...] * wt_ref[...] + bt_ref[...]


def _dense_tc(condition_emb, W_cond, b_cond, t, W_t, b_t):
    return pl.pallas_call(
        _dense_body,
        out_shape=(
            jax.ShapeDtypeStruct((B, D), jnp.float32),
            jax.ShapeDtypeStruct((B, D), jnp.float32),
        ),
    )(condition_emb, W_cond, b_cond.reshape(1, D), t.reshape(B, 1),
      W_t, b_t.reshape(1, D))


def _sc_gather(x2, temb, emb_table):
    mesh = plsc.VectorSubcoreMesh(core_axis_name="c", subcore_axis_name="s")

    @functools.partial(
        pl.kernel,
        mesh=mesh,
        compiler_params=pltpu.CompilerParams(use_tc_tiling_on_sc=False),
        out_type=jax.ShapeDtypeStruct((B, S1, D), jnp.float32),
        scratch_types=[
            pltpu.VMEM((2 * BPW, C0), jnp.int32),
            pltpu.VMEM((2, S, D), jnp.float32),
            pltpu.VMEM((BPW, 1, D), jnp.float32),
            pltpu.SemaphoreType.DMA,
            pltpu.SemaphoreType.DMA,
        ],
    )
    def k(x2_hbm, temb_hbm, table_hbm, out_hbm, idx_v, buf, tv, sem0, sem1):
        wid = lax.axis_index("s") * NC + lax.axis_index("c")
        base = wid * BPW
        sems = (sem0, sem1)

        pltpu.sync_copy(x2_hbm.at[pl.ds(base * 2, 2 * BPW)], idx_v)
        pltpu.sync_copy(temb_hbm.at[pl.ds(base, BPW)], tv)
        pltpu.sync_copy(tv, out_hbm.at[pl.ds(base, BPW), pl.ds(0, 1)])

        @pl.loop(0, BPW, step=2)
        def body(i):
            descs = []
            for kk in range(2):
                b = i + kk
                d1 = pltpu.async_copy(
                    table_hbm.at[idx_v.at[2 * b]],
                    buf.at[kk, pl.ds(0, C0)], sems[kk])
                d2 = pltpu.async_copy(
                    table_hbm.at[idx_v.at[2 * b + 1, pl.ds(0, C1)]],
                    buf.at[kk, pl.ds(C0, C1)], sems[kk])
                descs.append((d1, d2))
            for kk in range(2):
                b = i + kk
                d1, d2 = descs[kk]
                d1.wait()
                d2.wait()
                pltpu.sync_copy(buf.at[kk], out_hbm.at[base + b, pl.ds(1, S)])

    return k(x2, temb, emb_table)


def kernel(x, t, condition_emb, emb_table, W_cond, b_cond, W_t, b_t):
    cond, temb = _dense_tc(condition_emb, W_cond, b_cond, t, W_t, b_t)
    x2 = jnp.pad(x, ((0, 0), (0, 2 * C0 - S))).reshape(2 * B, C0)
    dit = _sc_gather(x2, temb.reshape(B, 1, D), emb_table)
    return (dit, cond)


# speedup vs baseline: 1.0306x; 1.0306x over previous
import functools

import jax
import jax.numpy as jnp
from jax import lax
from jax.experimental import pallas as pl
from jax.experimental.pallas import tpu as pltpu
from jax.experimental.pallas import tpu_sc as plsc

B = 4096
S = 200
S1 = S + 1
D = 64
COND_DIM = 128
VOCAB = 1000000

NC = 2
NS = 16
NW = NC * NS
BPW = B // NW

C0 = 128
C1 = S - C0
NBUF = 4


def _dense_body(ce_ref, wc_ref, bc_ref, t_ref, wt_ref, bt_ref,
                cond_ref, temb_ref):
    cond_ref[...] = (
        jnp.dot(ce_ref[...], wc_ref[...], preferred_element_type=jnp.float32)
        + bc_ref[...]
    )
    temb_ref[...] = t_ref[...] * wt_ref[...] + bt_ref[...]


def _dense_tc(condition_emb, W_cond, b_cond, t, W_t, b_t):
    return pl.pallas_call(
        _dense_body,
        out_shape=(
            jax.ShapeDtypeStruct((B, D), jnp.float32),
            jax.ShapeDtypeStruct((B, D), jnp.float32),
        ),
    )(condition_emb, W_cond, b_cond.reshape(1, D), t.reshape(B, 1),
      W_t, b_t.reshape(1, D))


def _sc_gather(x2, temb, emb_table):
    mesh = plsc.VectorSubcoreMesh(core_axis_name="c", subcore_axis_name="s")

    @functools.partial(
        pl.kernel,
        mesh=mesh,
        compiler_params=pltpu.CompilerParams(use_tc_tiling_on_sc=False),
        out_type=jax.ShapeDtypeStruct((B, S1, D), jnp.float32),
        scratch_types=[
            pltpu.VMEM((2 * BPW, C0), jnp.int32),
            pltpu.VMEM((NBUF, S, D), jnp.float32),
            pltpu.VMEM((BPW, 1, D), jnp.float32),
            [pltpu.SemaphoreType.DMA] * NBUF,
        ],
    )
    def k(x2_hbm, temb_hbm, table_hbm, out_hbm, idx_v, buf, tv, sems):
        wid = lax.axis_index("s") * NC + lax.axis_index("c")
        base = wid * BPW

        pltpu.sync_copy(x2_hbm.at[pl.ds(base * 2, 2 * BPW)], idx_v)
        pltpu.sync_copy(temb_hbm.at[pl.ds(base, BPW)], tv)
        pltpu.sync_copy(tv, out_hbm.at[pl.ds(base, BPW), pl.ds(0, 1)])

        def issue(b, kk):
            pltpu.async_copy(
                table_hbm.at[idx_v.at[2 * b]],
                buf.at[kk, pl.ds(0, C0)], sems[kk])
            pltpu.async_copy(
                table_hbm.at[idx_v.at[2 * b + 1, pl.ds(0, C1)]],
                buf.at[kk, pl.ds(C0, C1)], sems[kk])

        for kk in range(NBUF):
            issue(kk, kk)

        @pl.loop(0, BPW, step=NBUF)
        def body(i):
            for kk in range(NBUF):
                b = i + kk
                pltpu.make_async_copy(
                    table_hbm.at[pl.ds(0, S)], buf.at[kk], sems[kk]).wait()
                pltpu.sync_copy(buf.at[kk], out_hbm.at[base + b, pl.ds(1, S)])
                nb = b + NBUF

                @pl.when(nb < BPW)
                def _():
                    issue(nb, kk)

    return k(x2, temb, emb_table)


def kernel(x, t, condition_emb, emb_table, W_cond, b_cond, W_t, b_t):
    cond, temb = _dense_tc(condition_emb, W_cond, b_cond, t, W_t, b_t)
    x2 = jnp.pad(x, ((0, 0), (0, 2 * C0 - S))).reshape(2 * B, C0)
    dit = _sc_gather(x2, temb.reshape(B, 1, D), emb_table)
    return (dit, cond)
